# Initial kernel scaffold; baseline (speedup 1.0000x reference)
import jax
import jax.numpy as jnp
from jax.experimental import pallas as pl


def kernel(node_feats, edge_feats, edge_index, W_proj, b_proj, We1, be1, We2, be2, Wc0a, bc0a, Wc0b, bc0b, Wc1a, bc1a, Wc1b, bc1b):
    raise NotImplementedError("write your pallas kernel here")


# speedup vs baseline: 2.5243x; 2.5243x over previous
import functools

import jax
import jax.numpy as jnp
from jax import lax
from jax.experimental import pallas as pl
from jax.experimental.pallas import tpu as pltpu
from jax.experimental.pallas import tpu_sc as plsc

_NC = 2
_NS = 16
_C = 80


def _proj_body(x_ref, w_ref, b_ref, o_ref):
    t = jnp.dot(x_ref[...], w_ref[...], preferred_element_type=jnp.float32)
    o_ref[...] = jnp.maximum(t + b_ref[...], 0.0)


def _edge_body(ef_ref, w1_ref, b1_ref, w2_ref, b2_ref, o_ref):
    t = jnp.dot(ef_ref[...], w1_ref[...], preferred_element_type=jnp.float32)
    t = jnp.maximum(t + b1_ref[...], 0.0)
    o_ref[...] = jnp.dot(t, w2_ref[...], preferred_element_type=jnp.float32) + b2_ref[...]


def _layer_body(h_ref, a0_ref, a1_ref, wa_ref, ba_ref, wb_ref, bb_ref, o_ref):
    t = h_ref[...] + a0_ref[...] + a1_ref[...]
    u = jnp.dot(t, wa_ref[...], preferred_element_type=jnp.float32)
    u = jnp.maximum(u + ba_ref[...], 0.0)
    v = jnp.dot(u, wb_ref[...], preferred_element_type=jnp.float32)
    o_ref[...] = jnp.maximum(v + bb_ref[...], 0.0)


def _proj(x, w, b, blk):
    n, d = x.shape
    grid = (n // blk,)
    return pl.pallas_call(
        _proj_body,
        grid=grid,
        in_specs=[
            pl.BlockSpec((blk, d), lambda i: (i, 0)),
            pl.BlockSpec((d, d), lambda i: (0, 0)),
            pl.BlockSpec((1, d), lambda i: (0, 0)),
        ],
        out_specs=pl.BlockSpec((blk, d), lambda i: (i, 0)),
        out_shape=jax.ShapeDtypeStruct((n, d), jnp.float32),
    )(x, w, b)


def _edge_mlp(ef, w1, b1, w2, b2, blk):
    e_cnt, de = ef.shape
    d = w1.shape[1]
    grid = (e_cnt // blk,)
    return pl.pallas_call(
        _edge_body,
        grid=grid,
        in_specs=[
            pl.BlockSpec((blk, de), lambda i: (i, 0)),
            pl.BlockSpec((de, d), lambda i: (0, 0)),
            pl.BlockSpec((1, d), lambda i: (0, 0)),
            pl.BlockSpec((d, d), lambda i: (0, 0)),
            pl.BlockSpec((1, d), lambda i: (0, 0)),
        ],
        out_specs=pl.BlockSpec((blk, d), lambda i: (i, 0)),
        out_shape=jax.ShapeDtypeStruct((e_cnt, d), jnp.float32),
    )(ef, w1, b1, w2, b2)


def _layer(h, a0, a1, wa, ba, wb, bb, blk):
    n, d = h.shape
    grid = (n // blk,)
    return pl.pallas_call(
        _layer_body,
        grid=grid,
        in_specs=[
            pl.BlockSpec((blk, d), lambda i: (i, 0)),
            pl.BlockSpec((blk, d), lambda i: (i, 0)),
            pl.BlockSpec((blk, d), lambda i: (i, 0)),
            pl.BlockSpec((d, d), lambda i: (0, 0)),
            pl.BlockSpec((1, d), lambda i: (0, 0)),
            pl.BlockSpec((d, d), lambda i: (0, 0)),
            pl.BlockSpec((1, d), lambda i: (0, 0)),
        ],
        out_specs=pl.BlockSpec((blk, d), lambda i: (i, 0)),
        out_shape=jax.ShapeDtypeStruct((n, d), jnp.float32),
    )(h, a0, a1, wa, ba, wb, bb)


@functools.cache
def _make_sc_aggr(n, d, e_cnt):
    nw = _NC * _NS
    epw = e_cnt // nw
    chunks = epw // _C
    rows_pt = (n // _NS) // 8 * 8
    tail_off = rows_pt * _NS
    tail = n - tail_off
    assert epw * nw == e_cnt and chunks * _C == epw and tail % 8 == 0 and tail_off % 8 == 0
    mesh = plsc.VectorSubcoreMesh(core_axis_name="c", subcore_axis_name="s")

    @functools.partial(
        pl.kernel,
        out_type=jax.ShapeDtypeStruct((_NC, n, d), jnp.float32),
        mesh=mesh,
        scratch_types=[
            pltpu.VMEM((_C,), jnp.int32),
            pltpu.VMEM((_C,), jnp.int32),
            pltpu.VMEM((_C, d), jnp.float32),
            pltpu.VMEM((_C, d), jnp.float32),
            pltpu.VMEM_SHARED((n, d), jnp.float32),
            pltpu.SemaphoreType.DMA,
        ],
    )
    def sc_aggr(h_hbm, e_hbm, src_hbm, dst_hbm, z_hbm, out_hbm,
                src_v, dst_v, rows_v, e_v, aggr_sh, sem):
        cid = lax.axis_index("c")
        sid = lax.axis_index("s")
        wid = cid * _NS + sid
        base = wid * epw

        pltpu.sync_copy(z_hbm.at[pl.ds(sid * rows_pt, rows_pt)],
                        aggr_sh.at[pl.ds(sid * rows_pt, rows_pt)])
        if tail:
            @pl.when(sid == _NS - 1)
            def _zero_tail():
                pltpu.sync_copy(z_hbm.at[pl.ds(tail_off, tail)],
                                aggr_sh.at[pl.ds(tail_off, tail)])
        plsc.subcore_barrier()

        def chunk(i, carry):
            off = base + i * _C
            pltpu.sync_copy(src_hbm.at[pl.ds(off, _C)], src_v)
            pltpu.sync_copy(dst_hbm.at[pl.ds(off, _C)], dst_v)
            pltpu.async_copy(h_hbm.at[src_v], rows_v, sem).wait()
            pltpu.sync_copy(e_hbm.at[pl.ds(off, _C)], e_v)

            def row(r, c2):
                for j in range(d // 16):
                    s = pl.ds(j * 16, 16)
                    rows_v[r, s] = jnp.maximum(rows_v[r, s] + e_v[r, s], 0.0)
                return c2
            lax.fori_loop(0, _C, row, 0)

            pltpu.sync_copy(rows_v, aggr_sh.at[dst_v], add=True)
            return carry

        lax.fori_loop(0, chunks, chunk, 0)

        plsc.subcore_barrier()
        pltpu.sync_copy(aggr_sh.at[pl.ds(sid * rows_pt, rows_pt)],
                        out_hbm.at[cid, pl.ds(sid * rows_pt, rows_pt)])
        if tail:
            @pl.when(sid == _NS - 1)
            def _drain_tail():
                pltpu.sync_copy(aggr_sh.at[pl.ds(tail_off, tail)],
                                out_hbm.at[cid, pl.ds(tail_off, tail)])

    return sc_aggr


def kernel(node_feats, edge_feats, edge_index, W_proj, b_proj, We1, be1,
           We2, be2, Wc0a, bc0a, Wc0b, bc0b, Wc1a, bc1a, Wc1b, bc1b):
    n, d = node_feats.shape
    e_cnt = edge_feats.shape[0]
    src = edge_index[0]
    dst = edge_index[1]
    zeros_nd = jnp.zeros((n, d), jnp.float32)

    h = _proj(node_feats, W_proj, b_proj.reshape(1, d), blk=2000)
    e = _edge_mlp(edge_feats, We1, be1.reshape(1, d), We2, be2.reshape(1, d),
                  blk=2560)

    sc_aggr = _make_sc_aggr(n, d, e_cnt)
    for (wa, ba, wb, bb) in ((Wc0a, bc0a, Wc0b, bc0b), (Wc1a, bc1a, Wc1b, bc1b)):
        agg = sc_aggr(h, e, src, dst, zeros_nd)
        h = _layer(h, agg[0], agg[1], wa, ba.reshape(1, d), wb, bb.reshape(1, d),
                   blk=2000)
    return h


# speedup vs baseline: 2.9102x; 1.1529x over previous
import functools

import jax
import jax.numpy as jnp
from jax import lax
from jax.experimental import pallas as pl
from jax.experimental.pallas import tpu as pltpu
from jax.experimental.pallas import tpu_sc as plsc

_NC = 2
_NS = 16
_C = 48


def _proj_body(x_ref, w_ref, b_ref, o_ref):
    t = jnp.dot(x_ref[...], w_ref[...], preferred_element_type=jnp.float32)
    o_ref[...] = jnp.maximum(t + b_ref[...], 0.0)


def _edge_body(ef_ref, w1_ref, b1_ref, w2_ref, b2_ref, o_ref):
    t = jnp.dot(ef_ref[...], w1_ref[...], preferred_element_type=jnp.float32)
    t = jnp.maximum(t + b1_ref[...], 0.0)
    o_ref[...] = jnp.dot(t, w2_ref[...], preferred_element_type=jnp.float32) + b2_ref[...]


def _layer_body(h_ref, a0_ref, a1_ref, wa_ref, ba_ref, wb_ref, bb_ref, o_ref):
    t = h_ref[...] + a0_ref[...] + a1_ref[...]
    u = jnp.dot(t, wa_ref[...], preferred_element_type=jnp.float32)
    u = jnp.maximum(u + ba_ref[...], 0.0)
    v = jnp.dot(u, wb_ref[...], preferred_element_type=jnp.float32)
    o_ref[...] = jnp.maximum(v + bb_ref[...], 0.0)


def _proj(x, w, b, blk):
    n, d = x.shape
    grid = (n // blk,)
    return pl.pallas_call(
        _proj_body,
        grid=grid,
        in_specs=[
            pl.BlockSpec((blk, d), lambda i: (i, 0)),
            pl.BlockSpec((d, d), lambda i: (0, 0)),
            pl.BlockSpec((1, d), lambda i: (0, 0)),
        ],
        out_specs=pl.BlockSpec((blk, d), lambda i: (i, 0)),
        out_shape=jax.ShapeDtypeStruct((n, d), jnp.float32),
    )(x, w, b)


def _edge_mlp(ef, w1, b1, w2, b2, blk):
    e_cnt, de = ef.shape
    d = w1.shape[1]
    grid = (e_cnt // blk,)
    return pl.pallas_call(
        _edge_body,
        grid=grid,
        in_specs=[
            pl.BlockSpec((blk, de), lambda i: (i, 0)),
            pl.BlockSpec((de, d), lambda i: (0, 0)),
            pl.BlockSpec((1, d), lambda i: (0, 0)),
            pl.BlockSpec((d, d), lambda i: (0, 0)),
            pl.BlockSpec((1, d), lambda i: (0, 0)),
        ],
        out_specs=pl.BlockSpec((blk, d), lambda i: (i, 0)),
        out_shape=jax.ShapeDtypeStruct((e_cnt, d), jnp.float32),
    )(ef, w1, b1, w2, b2)


def _layer(h, a0, a1, wa, ba, wb, bb, blk):
    n, d = h.shape
    grid = (n // blk,)
    return pl.pallas_call(
        _layer_body,
        grid=grid,
        in_specs=[
            pl.BlockSpec((blk, d), lambda i: (i, 0)),
            pl.BlockSpec((blk, d), lambda i: (i, 0)),
            pl.BlockSpec((blk, d), lambda i: (i, 0)),
            pl.BlockSpec((d, d), lambda i: (0, 0)),
            pl.BlockSpec((1, d), lambda i: (0, 0)),
            pl.BlockSpec((d, d), lambda i: (0, 0)),
            pl.BlockSpec((1, d), lambda i: (0, 0)),
        ],
        out_specs=pl.BlockSpec((blk, d), lambda i: (i, 0)),
        out_shape=jax.ShapeDtypeStruct((n, d), jnp.float32),
    )(h, a0, a1, wa, ba, wb, bb)


@functools.cache
def _make_sc_aggr(n, d, e_cnt):
    nw = _NC * _NS
    epw = e_cnt // nw
    C = _C
    chunks = (epw // C) // 3 * 3
    tail_e = epw - chunks * C
    rows_pt = (n // _NS) // 8 * 8
    tail_off = rows_pt * _NS
    tail_n = n - tail_off
    assert epw * nw == e_cnt and tail_e % 16 == 0
    assert tail_n % 8 == 0 and tail_off % 8 == 0
    mesh = plsc.VectorSubcoreMesh(core_axis_name="c", subcore_axis_name="s")
    vm = pltpu.VMEM

    @functools.partial(
        pl.kernel,
        out_type=jax.ShapeDtypeStruct((_NC, n, d), jnp.float32),
        mesh=mesh,
        scratch_types=[
            vm((epw,), jnp.int32),
            vm((C, d), jnp.float32), vm((C, d), jnp.float32), vm((C, d), jnp.float32),
            vm((C, d), jnp.float32), vm((C, d), jnp.float32), vm((C, d), jnp.float32),
            vm((C,), jnp.int32), vm((C,), jnp.int32), vm((C,), jnp.int32),
            vm((16,), jnp.int32),
            pltpu.VMEM_SHARED((n, d), jnp.float32),
            pltpu.SemaphoreType.DMA, pltpu.SemaphoreType.DMA, pltpu.SemaphoreType.DMA,
            pltpu.SemaphoreType.DMA, pltpu.SemaphoreType.DMA, pltpu.SemaphoreType.DMA,
        ],
    )
    def sc_aggr(h_hbm, e_hbm, src_hbm, dst_hbm, z_hbm, out_hbm,
                s_all, r0, r1, r2, e0, e1, e2, dv0, dv1, dv2, dtt,
                aggr_sh, sd0, sd1, sd2, ss0, ss1, ss2):
        rows = (r0, r1, r2)
        evs = (e0, e1, e2)
        dvs = (dv0, dv1, dv2)
        sds = (sd0, sd1, sd2)
        sss = (ss0, ss1, ss2)
        cid = lax.axis_index("c")
        sid = lax.axis_index("s")
        wid = cid * _NS + sid
        base = wid * epw

        pltpu.sync_copy(src_hbm.at[pl.ds(base, epw)], s_all)
        pltpu.sync_copy(z_hbm.at[pl.ds(sid * rows_pt, rows_pt)],
                        aggr_sh.at[pl.ds(sid * rows_pt, rows_pt)])
        if tail_n:
            @pl.when(sid == _NS - 1)
            def _zero_tail():
                pltpu.sync_copy(z_hbm.at[pl.ds(tail_off, tail_n)],
                                aggr_sh.at[pl.ds(tail_off, tail_n)])
        plsc.subcore_barrier()

        def dat_descs(i, b):
            g = pltpu.make_async_copy(h_hbm.at[s_all.at[pl.ds(i * C, C)]],
                                      rows[b], sds[b])
            s = pltpu.make_async_copy(e_hbm.at[pl.ds(base + i * C, C)],
                                      evs[b], sds[b])
            t = pltpu.make_async_copy(dst_hbm.at[pl.ds(base + i * C, C)],
                                      dvs[b], sds[b])
            return g, s, t

        def fire_dat(i, b):
            for cp in dat_descs(i, b):
                cp.start()

        fire_dat(0, 0)
        fire_dat(1, 1)

        @pl.loop(0, chunks, step=3)
        def grp(g0):
            for b in range(3):
                i = g0 + b
                b2 = (b + 2) % 3
                for cp in dat_descs(i, b):
                    cp.wait()
                R = rows[b]
                Ebuf = evs[b]

                @pl.loop(0, C, unroll=4)
                def row(r):
                    for j in range(d // 16):
                        s = pl.ds(j * 16, 16)
                        R[r, s] = jnp.maximum(R[r, s] + Ebuf[r, s], 0.0)

                pltpu.async_copy(R, aggr_sh.at[dvs[b]], sss[b], add=True)

                @pl.when(i >= 1)
                def _wait_prev_scatter():
                    pltpu.make_async_copy(rows[b2], aggr_sh.at[dvs[b2]],
                                          sss[b2]).wait()

                @pl.when(i + 2 < chunks)
                def _fire_ahead():
                    fire_dat(i + 2, b2)

        bl = (chunks - 1) % 3
        pltpu.make_async_copy(rows[bl], aggr_sh.at[dvs[bl]], sss[bl]).wait()

        for t in range(tail_e // 16):
            toff = chunks * C + t * 16
            pltpu.async_copy(h_hbm.at[s_all.at[pl.ds(toff, 16)]],
                             r0.at[pl.ds(0, 16)], sd0).wait()
            pltpu.sync_copy(e_hbm.at[pl.ds(base + toff, 16)],
                            e0.at[pl.ds(0, 16)])
            pltpu.sync_copy(dst_hbm.at[pl.ds(base + toff, 16)], dtt)

            @pl.loop(0, 16)
            def trow(r):
                for j in range(d // 16):
                    s = pl.ds(j * 16, 16)
                    r0[r, s] = jnp.maximum(r0[r, s] + e0[r, s], 0.0)

            pltpu.sync_copy(r0.at[pl.ds(0, 16)], aggr_sh.at[dtt], add=True)

        plsc.subcore_barrier()
        pltpu.sync_copy(aggr_sh.at[pl.ds(sid * rows_pt, rows_pt)],
                        out_hbm.at[cid, pl.ds(sid * rows_pt, rows_pt)])
        if tail_n:
            @pl.when(sid == _NS - 1)
            def _drain_tail():
                pltpu.sync_copy(aggr_sh.at[pl.ds(tail_off, tail_n)],
                                out_hbm.at[cid, pl.ds(tail_off, tail_n)])

    return sc_aggr


def kernel(node_feats, edge_feats, edge_index, W_proj, b_proj, We1, be1,
           We2, be2, Wc0a, bc0a, Wc0b, bc0b, Wc1a, bc1a, Wc1b, bc1b):
    n, d = node_feats.shape
    e_cnt = edge_feats.shape[0]
    src = edge_index[0]
    dst = edge_index[1]
    zeros_nd = jnp.zeros((n, d), jnp.float32)

    h = _proj(node_feats, W_proj, b_proj.reshape(1, d), blk=2000)
    e = _edge_mlp(edge_feats, We1, be1.reshape(1, d), We2, be2.reshape(1, d),
                  blk=2560)

    sc_aggr = _make_sc_aggr(n, d, e_cnt)
    for (wa, ba, wb, bb) in ((Wc0a, bc0a, Wc0b, bc0b), (Wc1a, bc1a, Wc1b, bc1b)):
        agg = sc_aggr(h, e, src, dst, zeros_nd)
        h = _layer(h, agg[0], agg[1], wa, ba.reshape(1, d), wb, bb.reshape(1, d),
                   blk=2000)
    return h


# speedup vs baseline: 4.6972x; 1.6140x over previous
import functools

import jax
import jax.numpy as jnp
from jax import lax
from jax.experimental import pallas as pl
from jax.experimental.pallas import tpu as pltpu
from jax.experimental.pallas import tpu_sc as plsc

_NC = 2
_NS = 16
_C = 48


def _proj_body(x_ref, w_ref, b_ref, o_ref):
    t = jnp.dot(x_ref[...], w_ref[...], preferred_element_type=jnp.float32)
    o_ref[...] = jnp.maximum(t + b_ref[...], 0.0)


def _edge_body(ef_ref, w1_ref, b1_ref, w2_ref, b2_ref, o_ref):
    t = jnp.dot(ef_ref[...], w1_ref[...], preferred_element_type=jnp.float32)
    t = jnp.maximum(t + b1_ref[...], 0.0)
    o_ref[...] = jnp.dot(t, w2_ref[...], preferred_element_type=jnp.float32) + b2_ref[...]


def _layer_body(h_ref, a0_ref, a1_ref, wa_ref, ba_ref, wb_ref, bb_ref, o_ref):
    t = h_ref[...] + a0_ref[...] + a1_ref[...]
    u = jnp.dot(t, wa_ref[...], preferred_element_type=jnp.float32)
    u = jnp.maximum(u + ba_ref[...], 0.0)
    v = jnp.dot(u, wb_ref[...], preferred_element_type=jnp.float32)
    o_ref[...] = jnp.maximum(v + bb_ref[...], 0.0)


def _proj(x, w, b, blk):
    n, d = x.shape
    grid = (n // blk,)
    return pl.pallas_call(
        _proj_body,
        grid=grid,
        in_specs=[
            pl.BlockSpec((blk, d), lambda i: (i, 0)),
            pl.BlockSpec((d, d), lambda i: (0, 0)),
            pl.BlockSpec((1, d), lambda i: (0, 0)),
        ],
        out_specs=pl.BlockSpec((blk, d), lambda i: (i, 0)),
        out_shape=jax.ShapeDtypeStruct((n, d), jnp.float32),
    )(x, w, b)


def _edge_mlp(ef, w1, b1, w2, b2, blk):
    e_cnt, de = ef.shape
    d = w1.shape[1]
    grid = (e_cnt // blk,)
    return pl.pallas_call(
        _edge_body,
        grid=grid,
        in_specs=[
            pl.BlockSpec((blk, de), lambda i: (i, 0)),
            pl.BlockSpec((de, d), lambda i: (0, 0)),
            pl.BlockSpec((1, d), lambda i: (0, 0)),
            pl.BlockSpec((d, d), lambda i: (0, 0)),
            pl.BlockSpec((1, d), lambda i: (0, 0)),
        ],
        out_specs=pl.BlockSpec((blk, d), lambda i: (i, 0)),
        out_shape=jax.ShapeDtypeStruct((e_cnt, d), jnp.float32),
    )(ef, w1, b1, w2, b2)


def _layer(h, a0, a1, wa, ba, wb, bb, blk):
    n, d = h.shape
    grid = (n // blk,)
    return pl.pallas_call(
        _layer_body,
        grid=grid,
        in_specs=[
            pl.BlockSpec((blk, d), lambda i: (i, 0)),
            pl.BlockSpec((blk, d), lambda i: (i, 0)),
            pl.BlockSpec((blk, d), lambda i: (i, 0)),
            pl.BlockSpec((d, d), lambda i: (0, 0)),
            pl.BlockSpec((1, d), lambda i: (0, 0)),
            pl.BlockSpec((d, d), lambda i: (0, 0)),
            pl.BlockSpec((1, d), lambda i: (0, 0)),
        ],
        out_specs=pl.BlockSpec((blk, d), lambda i: (i, 0)),
        out_shape=jax.ShapeDtypeStruct((n, d), jnp.float32),
    )(h, a0, a1, wa, ba, wb, bb)


@functools.cache
def _make_sc_aggr(n, d, e_cnt):
    nw = _NC * _NS
    epw = e_cnt // nw
    C = _C
    chunks = (epw // C) // 3 * 3
    tail_e = epw - chunks * C
    rows_pt = (n // _NS) // 8 * 8
    tail_off = rows_pt * _NS
    tail_n = n - tail_off
    assert epw * nw == e_cnt and tail_e % 16 == 0
    assert tail_n % 8 == 0 and tail_off % 8 == 0
    mesh = plsc.VectorSubcoreMesh(core_axis_name="c", subcore_axis_name="s")
    vm = pltpu.VMEM

    @functools.partial(
        pl.kernel,
        out_type=jax.ShapeDtypeStruct((_NC, n, d), jnp.float32),
        mesh=mesh,
        scratch_types=[
            vm((epw,), jnp.int32),
            vm((C, d), jnp.float32), vm((C, d), jnp.float32), vm((C, d), jnp.float32),
            vm((C, d), jnp.float32), vm((C, d), jnp.float32), vm((C, d), jnp.float32),
            vm((C,), jnp.int32), vm((C,), jnp.int32), vm((C,), jnp.int32),
            vm((16,), jnp.int32),
            pltpu.VMEM_SHARED((n, d), jnp.float32),
            pltpu.SemaphoreType.DMA, pltpu.SemaphoreType.DMA, pltpu.SemaphoreType.DMA,
            pltpu.SemaphoreType.DMA, pltpu.SemaphoreType.DMA, pltpu.SemaphoreType.DMA,
        ],
    )
    def sc_aggr(h_hbm, e_hbm, src_hbm, dst_hbm, z_hbm, out_hbm,
                s_all, r0, r1, r2, e0, e1, e2, dv0, dv1, dv2, dtt,
                aggr_sh, sd0, sd1, sd2, ss0, ss1, ss2):
        rows = (r0, r1, r2)
        evs = (e0, e1, e2)
        dvs = (dv0, dv1, dv2)
        sds = (sd0, sd1, sd2)
        sss = (ss0, ss1, ss2)
        cid = lax.axis_index("c")
        sid = lax.axis_index("s")
        wid = cid * _NS + sid
        base = wid * epw

        pltpu.sync_copy(src_hbm.at[pl.ds(base, epw)], s_all)
        pltpu.sync_copy(z_hbm.at[pl.ds(sid * rows_pt, rows_pt)],
                        aggr_sh.at[pl.ds(sid * rows_pt, rows_pt)])
        if tail_n:
            @pl.when(sid == _NS - 1)
            def _zero_tail():
                pltpu.sync_copy(z_hbm.at[pl.ds(tail_off, tail_n)],
                                aggr_sh.at[pl.ds(tail_off, tail_n)])
        plsc.subcore_barrier()

        def dat_descs(i, b):
            g = pltpu.make_async_copy(h_hbm.at[s_all.at[pl.ds(i * C, C)]],
                                      rows[b], sds[b])
            s = pltpu.make_async_copy(e_hbm.at[pl.ds(base + i * C, C)],
                                      evs[b], sds[b])
            t = pltpu.make_async_copy(dst_hbm.at[pl.ds(base + i * C, C)],
                                      dvs[b], sds[b])
            return g, s, t

        def fire_dat(i, b):
            for cp in dat_descs(i, b):
                cp.start()

        fire_dat(0, 0)
        fire_dat(1, 1)

        @pl.loop(0, chunks, step=3)
        def grp(g0):
            for b in range(3):
                i = g0 + b
                b2 = (b + 2) % 3
                for cp in dat_descs(i, b):
                    cp.wait()
                R = rows[b]
                Ebuf = evs[b]

                @plsc.parallel_loop(0, C, 1, unroll=8)
                def row(r):
                    for j in range(d // 16):
                        s = pl.ds(j * 16, 16)
                        R[r, s] = jnp.maximum(R[r, s] + Ebuf[r, s], 0.0)

                pltpu.async_copy(R, aggr_sh.at[dvs[b]], sss[b], add=True)

                @pl.when(i >= 1)
                def _wait_prev_scatter():
                    pltpu.make_async_copy(rows[b2], aggr_sh.at[dvs[b2]],
                                          sss[b2]).wait()

                @pl.when(i + 2 < chunks)
                def _fire_ahead():
                    fire_dat(i + 2, b2)

        bl = (chunks - 1) % 3
        pltpu.make_async_copy(rows[bl], aggr_sh.at[dvs[bl]], sss[bl]).wait()

        for t in range(tail_e // 16):
            toff = chunks * C + t * 16
            pltpu.async_copy(h_hbm.at[s_all.at[pl.ds(toff, 16)]],
                             r0.at[pl.ds(0, 16)], sd0).wait()
            pltpu.sync_copy(e_hbm.at[pl.ds(base + toff, 16)],
                            e0.at[pl.ds(0, 16)])
            pltpu.sync_copy(dst_hbm.at[pl.ds(base + toff, 16)], dtt)

            @pl.loop(0, 16)
            def trow(r):
                for j in range(d // 16):
                    s = pl.ds(j * 16, 16)
                    r0[r, s] = jnp.maximum(r0[r, s] + e0[r, s], 0.0)

            pltpu.sync_copy(r0.at[pl.ds(0, 16)], aggr_sh.at[dtt], add=True)

        plsc.subcore_barrier()
        pltpu.sync_copy(aggr_sh.at[pl.ds(sid * rows_pt, rows_pt)],
                        out_hbm.at[cid, pl.ds(sid * rows_pt, rows_pt)])
        if tail_n:
            @pl.when(sid == _NS - 1)
            def _drain_tail():
                pltpu.sync_copy(aggr_sh.at[pl.ds(tail_off, tail_n)],
                                out_hbm.at[cid, pl.ds(tail_off, tail_n)])

    return sc_aggr


def kernel(node_feats, edge_feats, edge_index, W_proj, b_proj, We1, be1,
           We2, be2, Wc0a, bc0a, Wc0b, bc0b, Wc1a, bc1a, Wc1b, bc1b):
    n, d = node_feats.shape
    e_cnt = edge_feats.shape[0]
    src = edge_index[0]
    dst = edge_index[1]
    zeros_nd = jnp.zeros((n, d), jnp.float32)

    h = _proj(node_feats, W_proj, b_proj.reshape(1, d), blk=2000)
    e = _edge_mlp(edge_feats, We1, be1.reshape(1, d), We2, be2.reshape(1, d),
                  blk=2560)

    sc_aggr = _make_sc_aggr(n, d, e_cnt)
    for (wa, ba, wb, bb) in ((Wc0a, bc0a, Wc0b, bc0b), (Wc1a, bc1a, Wc1b, bc1b)):
        agg = sc_aggr(h, e, src, dst, zeros_nd)
        h = _layer(h, agg[0], agg[1], wa, ba.reshape(1, d), wb, bb.reshape(1, d),
                   blk=2000)
    return h


# speedup vs baseline: 4.7135x; 1.0035x over previous
import functools

import jax
import jax.numpy as jnp
from jax import lax
from jax.experimental import pallas as pl
from jax.experimental.pallas import tpu as pltpu
from jax.experimental.pallas import tpu_sc as plsc

_NC = 2
_NS = 16
_C = 48
_R = 4


def _proj_body(x_ref, w_ref, b_ref, o_ref):
    t = jnp.dot(x_ref[...], w_ref[...], preferred_element_type=jnp.float32)
    o_ref[...] = jnp.maximum(t + b_ref[...], 0.0)


def _edge_body(ef_ref, w1_ref, b1_ref, w2_ref, b2_ref, o_ref):
    t = jnp.dot(ef_ref[...], w1_ref[...], preferred_element_type=jnp.float32)
    t = jnp.maximum(t + b1_ref[...], 0.0)
    o_ref[...] = jnp.dot(t, w2_ref[...], preferred_element_type=jnp.float32) + b2_ref[...]


def _layer_body(h_ref, a0_ref, a1_ref, wa_ref, ba_ref, wb_ref, bb_ref, o_ref):
    t = h_ref[...] + a0_ref[...] + a1_ref[...]
    u = jnp.dot(t, wa_ref[...], preferred_element_type=jnp.float32)
    u = jnp.maximum(u + ba_ref[...], 0.0)
    v = jnp.dot(u, wb_ref[...], preferred_element_type=jnp.float32)
    o_ref[...] = jnp.maximum(v + bb_ref[...], 0.0)


def _proj(x, w, b, blk):
    n, d = x.shape
    return pl.pallas_call(
        _proj_body,
        grid=(n // blk,),
        in_specs=[
            pl.BlockSpec((blk, d), lambda i: (i, 0)),
            pl.BlockSpec((d, d), lambda i: (0, 0)),
            pl.BlockSpec((1, d), lambda i: (0, 0)),
        ],
        out_specs=pl.BlockSpec((blk, d), lambda i: (i, 0)),
        out_shape=jax.ShapeDtypeStruct((n, d), jnp.float32),
    )(x, w, b)


def _edge_mlp(ef, w1, b1, w2, b2, blk):
    e_cnt, de = ef.shape
    d = w1.shape[1]
    return pl.pallas_call(
        _edge_body,
        grid=(e_cnt // blk,),
        in_specs=[
            pl.BlockSpec((blk, de), lambda i: (i, 0)),
            pl.BlockSpec((de, d), lambda i: (0, 0)),
            pl.BlockSpec((1, d), lambda i: (0, 0)),
            pl.BlockSpec((d, d), lambda i: (0, 0)),
            pl.BlockSpec((1, d), lambda i: (0, 0)),
        ],
        out_specs=pl.BlockSpec((blk, d), lambda i: (i, 0)),
        out_shape=jax.ShapeDtypeStruct((e_cnt, d), jnp.float32),
    )(ef, w1, b1, w2, b2)


def _layer(h, a0, a1, wa, ba, wb, bb, blk):
    n, d = h.shape
    return pl.pallas_call(
        _layer_body,
        grid=(n // blk,),
        in_specs=[
            pl.BlockSpec((blk, d), lambda i: (i, 0)),
            pl.BlockSpec((blk, d), lambda i: (i, 0)),
            pl.BlockSpec((blk, d), lambda i: (i, 0)),
            pl.BlockSpec((d, d), lambda i: (0, 0)),
            pl.BlockSpec((1, d), lambda i: (0, 0)),
            pl.BlockSpec((d, d), lambda i: (0, 0)),
            pl.BlockSpec((1, d), lambda i: (0, 0)),
        ],
        out_specs=pl.BlockSpec((blk, d), lambda i: (i, 0)),
        out_shape=jax.ShapeDtypeStruct((n, d), jnp.float32),
    )(h, a0, a1, wa, ba, wb, bb)


@functools.cache
def _make_sc_aggr(n, d, e_cnt):
    nw = _NC * _NS
    epw = e_cnt // nw
    C = _C
    R = _R
    chunks = (epw // C) // R * R
    tail_e = epw - chunks * C
    rows_pt = (n // _NS) // 8 * 8
    tail_off = rows_pt * _NS
    tail_n = n - tail_off
    assert epw * nw == e_cnt and tail_e % 16 == 0
    assert tail_n % 8 == 0 and tail_off % 8 == 0
    mesh = plsc.VectorSubcoreMesh(core_axis_name="c", subcore_axis_name="s")
    vm = pltpu.VMEM
    DMA = pltpu.SemaphoreType.DMA

    @functools.partial(
        pl.kernel,
        out_type=jax.ShapeDtypeStruct((_NC, n, d), jnp.float32),
        mesh=mesh,
        scratch_types=[
            [vm((C,), jnp.int32)] * R,
            [vm((C,), jnp.int32)] * R,
            [vm((C, d), jnp.float32)] * R,
            [vm((C, d), jnp.float32)] * R,
            vm((16,), jnp.int32),
            vm((16,), jnp.int32),
            pltpu.VMEM_SHARED((n, d), jnp.float32),
            [DMA] * R,
            [DMA] * R,
            [DMA] * R,
            [DMA] * R,
        ],
    )
    def sc_aggr(h_hbm, e_hbm, src_hbm, dst_hbm, z_hbm, out_hbm,
                svs, dvs, rows, evs, stt, dtt, aggr_sh, sis, sts, sds, sss):
        cid = lax.axis_index("c")
        sid = lax.axis_index("s")
        wid = cid * _NS + sid
        base = wid * epw

        def si_desc(i, b):
            return pltpu.make_async_copy(src_hbm.at[pl.ds(base + i * C, C)],
                                         svs[b], sis[b])

        def st_desc(i, b):
            return pltpu.make_async_copy(dst_hbm.at[pl.ds(base + i * C, C)],
                                         dvs[b], sts[b])

        def dat_descs(i, b):
            g = pltpu.make_async_copy(h_hbm.at[svs[b]], rows[b], sds[b])
            s = pltpu.make_async_copy(e_hbm.at[pl.ds(base + i * C, C)],
                                      evs[b], sds[b])
            return g, s

        def sca_desc(b):
            return pltpu.make_async_copy(rows[b], aggr_sh.at[dvs[b]], sss[b])

        pltpu.sync_copy(z_hbm.at[pl.ds(sid * rows_pt, rows_pt)],
                        aggr_sh.at[pl.ds(sid * rows_pt, rows_pt)])
        if tail_n:
            @pl.when(sid == _NS - 1)
            def _zero_tail():
                pltpu.sync_copy(z_hbm.at[pl.ds(tail_off, tail_n)],
                                aggr_sh.at[pl.ds(tail_off, tail_n)])
        plsc.subcore_barrier()

        for b in range(3):
            si_desc(b, b).start()
        st_desc(0, 0).start()
        si_desc(0, 0).wait()
        for cp in dat_descs(0, 0):
            cp.start()
        si_desc(1, 1).wait()
        for cp in dat_descs(1, 1):
            cp.start()

        @pl.loop(0, chunks, step=R)
        def grp(g0):
            for b in range(R):
                i = g0 + b
                b1 = (b + 1) % R
                b2 = (b + 2) % R
                b3 = (b + 3) % R
                for cp in dat_descs(i, b):
                    cp.wait()
                Rw = rows[b]
                Ev = evs[b]

                @plsc.parallel_loop(0, C, 1, unroll=8)
                def row(r):
                    for j in range(d // 16):
                        s = pl.ds(j * 16, 16)
                        Rw[r, s] = jnp.maximum(Rw[r, s] + Ev[r, s], 0.0)

                st_desc(i, b).wait()
                pltpu.async_copy(Rw, aggr_sh.at[dvs[b]], sss[b], add=True)

                @pl.when(i >= 2)
                def _wait_prev_scatter():
                    sca_desc(b2).wait()

                @pl.when(i + 2 < chunks)
                def _fire_data_ahead():
                    si_desc(i + 2, b2).wait()
                    for cp in dat_descs(i + 2, b2):
                        cp.start()

                @pl.when(i + 3 < chunks)
                def _fire_src_idx():
                    si_desc(i + 3, b3).start()

                @pl.when(i + 1 < chunks)
                def _fire_dst_idx():
                    st_desc(i + 1, b1).start()

        sca_desc((chunks - 2) % R).wait()
        sca_desc((chunks - 1) % R).wait()

        for t in range(tail_e // 16):
            toff = chunks * C + t * 16
            pltpu.sync_copy(src_hbm.at[pl.ds(base + toff, 16)], stt)
            pltpu.sync_copy(dst_hbm.at[pl.ds(base + toff, 16)], dtt)
            pltpu.async_copy(h_hbm.at[stt], rows[0].at[pl.ds(0, 16)],
                             sds[0]).wait()
            pltpu.sync_copy(e_hbm.at[pl.ds(base + toff, 16)],
                            evs[0].at[pl.ds(0, 16)])

            @pl.loop(0, 16)
            def trow(r):
                for j in range(d // 16):
                    s = pl.ds(j * 16, 16)
                    rows[0][r, s] = jnp.maximum(rows[0][r, s] + evs[0][r, s], 0.0)

            pltpu.sync_copy(rows[0].at[pl.ds(0, 16)], aggr_sh.at[dtt], add=True)

        plsc.subcore_barrier()
        pltpu.sync_copy(aggr_sh.at[pl.ds(sid * rows_pt, rows_pt)],
                        out_hbm.at[cid, pl.ds(sid * rows_pt, rows_pt)])
        if tail_n:
            @pl.when(sid == _NS - 1)
            def _drain_tail():
                pltpu.sync_copy(aggr_sh.at[pl.ds(tail_off, tail_n)],
                                out_hbm.at[cid, pl.ds(tail_off, tail_n)])

    return sc_aggr


def kernel(node_feats, edge_feats, edge_index, W_proj, b_proj, We1, be1,
           We2, be2, Wc0a, bc0a, Wc0b, bc0b, Wc1a, bc1a, Wc1b, bc1b):
    n, d = node_feats.shape
    e_cnt = edge_feats.shape[0]
    src = edge_index[0]
    dst = edge_index[1]
    zeros_nd = jnp.zeros((n, d), jnp.float32)

    h = _proj(node_feats, W_proj, b_proj.reshape(1, d), blk=2000)
    e = _edge_mlp(edge_feats, We1, be1.reshape(1, d), We2, be2.reshape(1, d),
                  blk=2560)

    sc_aggr = _make_sc_aggr(n, d, e_cnt)
    for (wa, ba, wb, bb) in ((Wc0a, bc0a, Wc0b, bc0b), (Wc1a, bc1a, Wc1b, bc1b)):
        agg = sc_aggr(h, e, src, dst, zeros_nd)
        h = _layer(h, agg[0], agg[1], wa, ba.reshape(1, d), wb, bb.reshape(1, d),
                   blk=2000)
    return h


# speedup vs baseline: 5.0686x; 1.0753x over previous
import functools

import jax
import jax.numpy as jnp
from jax import lax
from jax.experimental import pallas as pl
from jax.experimental.pallas import tpu as pltpu
from jax.experimental.pallas import tpu_sc as plsc

_NC = 2
_NS = 16
_C = 48
_R = 4


def _pack_halves(t):
    d2 = t.shape[1] // 2
    ti = jax.lax.bitcast_convert_type(t, jnp.int32)
    lo = jax.lax.shift_right_logical(ti[:, :d2] + jnp.int32(0x8000), 16)
    hi = (ti[:, d2:] + jnp.int32(0x8000)) & jnp.int32(-65536)
    return lo | hi


def _proj_body(x_ref, w_ref, b_ref, o_ref, op_ref):
    t = jnp.dot(x_ref[...], w_ref[...], preferred_element_type=jnp.float32)
    t = jnp.maximum(t + b_ref[...], 0.0)
    o_ref[...] = t
    op_ref[...] = _pack_halves(t)


def _edge_body(ef_ref, w1_ref, b1_ref, w2_ref, b2_ref, o_ref):
    t = jnp.dot(ef_ref[...], w1_ref[...], preferred_element_type=jnp.float32)
    t = jnp.maximum(t + b1_ref[...], 0.0)
    o_ref[...] = jnp.dot(t, w2_ref[...], preferred_element_type=jnp.float32) + b2_ref[...]


def _layer_mid_body(h_ref, a0_ref, a1_ref, wa_ref, ba_ref, wb_ref, bb_ref,
                    o_ref, op_ref):
    t = h_ref[...] + a0_ref[...] + a1_ref[...]
    u = jnp.dot(t, wa_ref[...], preferred_element_type=jnp.float32)
    u = jnp.maximum(u + ba_ref[...], 0.0)
    v = jnp.dot(u, wb_ref[...], preferred_element_type=jnp.float32)
    v = jnp.maximum(v + bb_ref[...], 0.0)
    o_ref[...] = v
    op_ref[...] = _pack_halves(v)


def _layer_out_body(h_ref, a0_ref, a1_ref, wa_ref, ba_ref, wb_ref, bb_ref, o_ref):
    t = h_ref[...] + a0_ref[...] + a1_ref[...]
    u = jnp.dot(t, wa_ref[...], preferred_element_type=jnp.float32)
    u = jnp.maximum(u + ba_ref[...], 0.0)
    v = jnp.dot(u, wb_ref[...], preferred_element_type=jnp.float32)
    o_ref[...] = jnp.maximum(v + bb_ref[...], 0.0)


def _proj(x, w, b, blk):
    n, d = x.shape
    return pl.pallas_call(
        _proj_body,
        grid=(n // blk,),
        in_specs=[
            pl.BlockSpec((blk, d), lambda i: (i, 0)),
            pl.BlockSpec((d, d), lambda i: (0, 0)),
            pl.BlockSpec((1, d), lambda i: (0, 0)),
        ],
        out_specs=[pl.BlockSpec((blk, d), lambda i: (i, 0)),
                   pl.BlockSpec((blk, d // 2), lambda i: (i, 0))],
        out_shape=[jax.ShapeDtypeStruct((n, d), jnp.float32),
                   jax.ShapeDtypeStruct((n, d // 2), jnp.int32)],
    )(x, w, b)


def _edge_mlp(ef, w1, b1, w2, b2, blk):
    e_cnt, de = ef.shape
    d = w1.shape[1]
    return pl.pallas_call(
        _edge_body,
        grid=(e_cnt // blk,),
        in_specs=[
            pl.BlockSpec((blk, de), lambda i: (i, 0)),
            pl.BlockSpec((de, d), lambda i: (0, 0)),
            pl.BlockSpec((1, d), lambda i: (0, 0)),
            pl.BlockSpec((d, d), lambda i: (0, 0)),
            pl.BlockSpec((1, d), lambda i: (0, 0)),
        ],
        out_specs=pl.BlockSpec((blk, d), lambda i: (i, 0)),
        out_shape=jax.ShapeDtypeStruct((e_cnt, d), jnp.float32),
    )(ef, w1, b1, w2, b2)


def _layer_specs(blk, d):
    return [
        pl.BlockSpec((blk, d), lambda i: (i, 0)),
        pl.BlockSpec((blk, d), lambda i: (i, 0)),
        pl.BlockSpec((blk, d), lambda i: (i, 0)),
        pl.BlockSpec((d, d), lambda i: (0, 0)),
        pl.BlockSpec((1, d), lambda i: (0, 0)),
        pl.BlockSpec((d, d), lambda i: (0, 0)),
        pl.BlockSpec((1, d), lambda i: (0, 0)),
    ]


def _layer_mid(h, a0, a1, wa, ba, wb, bb, blk):
    n, d = h.shape
    return pl.pallas_call(
        _layer_mid_body,
        grid=(n // blk,),
        in_specs=_layer_specs(blk, d),
        out_specs=[pl.BlockSpec((blk, d), lambda i: (i, 0)),
                   pl.BlockSpec((blk, d // 2), lambda i: (i, 0))],
        out_shape=[jax.ShapeDtypeStruct((n, d), jnp.float32),
                   jax.ShapeDtypeStruct((n, d // 2), jnp.int32)],
    )(h, a0, a1, wa, ba, wb, bb)


def _layer_out(h, a0, a1, wa, ba, wb, bb, blk):
    n, d = h.shape
    return pl.pallas_call(
        _layer_out_body,
        grid=(n // blk,),
        in_specs=_layer_specs(blk, d),
        out_specs=pl.BlockSpec((blk, d), lambda i: (i, 0)),
        out_shape=jax.ShapeDtypeStruct((n, d), jnp.float32),
    )(h, a0, a1, wa, ba, wb, bb)


def _f32_lo(w):
    return jax.lax.bitcast_convert_type(w << 16, jnp.float32)


def _f32_hi(w):
    return jax.lax.bitcast_convert_type(w & jnp.int32(-65536), jnp.float32)


@functools.cache
def _make_sc_aggr(n, d, e_cnt):
    nw = _NC * _NS
    epw = e_cnt // nw
    C = _C
    R = _R
    chunks = (epw // C) // R * R
    tail_e = epw - chunks * C
    rows_pt = (n // _NS) // 8 * 8
    tail_off = rows_pt * _NS
    tail_n = n - tail_off
    assert epw * nw == e_cnt and tail_e % 16 == 0
    assert tail_n % 8 == 0 and tail_off % 8 == 0
    mesh = plsc.VectorSubcoreMesh(core_axis_name="c", subcore_axis_name="s")
    vm = pltpu.VMEM
    DMA = pltpu.SemaphoreType.DMA

    @functools.partial(
        pl.kernel,
        out_type=jax.ShapeDtypeStruct((_NC, n, d), jnp.float32),
        mesh=mesh,
        compiler_params=pltpu.CompilerParams(use_tc_tiling_on_sc=False),
        scratch_types=[
            [vm((C,), jnp.int32)] * R,
            [vm((C,), jnp.int32)] * R,
            [vm((C, d // 2), jnp.int32)] * R,
            [vm((C, d), jnp.float32)] * R,
            vm((16,), jnp.int32),
            vm((16,), jnp.int32),
            pltpu.VMEM_SHARED((n, d), jnp.float32),
            [DMA] * R,
            [DMA] * R,
            [DMA] * R,
            [DMA] * R,
        ],
    )
    def sc_aggr(h_hbm, e_hbm, src_hbm, dst_hbm, z_hbm, out_hbm,
                svs, dvs, rows, evs, stt, dtt, aggr_sh, sis, sts, sds, sss):
        cid = lax.axis_index("c")
        sid = lax.axis_index("s")
        wid = cid * _NS + sid
        base = wid * epw

        def si_desc(i, b):
            return pltpu.make_async_copy(src_hbm.at[pl.ds(base + i * C, C)],
                                         svs[b], sis[b])

        def st_desc(i, b):
            return pltpu.make_async_copy(dst_hbm.at[pl.ds(base + i * C, C)],
                                         dvs[b], sts[b])

        def dat_descs(i, b):
            g = pltpu.make_async_copy(h_hbm.at[svs[b]], rows[b], sds[b])
            s = pltpu.make_async_copy(e_hbm.at[pl.ds(base + i * C, C)],
                                      evs[b], sds[b])
            return g, s

        def sca_desc(b):
            return pltpu.make_async_copy(evs[b], aggr_sh.at[dvs[b]], sss[b])

        pltpu.sync_copy(z_hbm.at[pl.ds(sid * rows_pt, rows_pt)],
                        aggr_sh.at[pl.ds(sid * rows_pt, rows_pt)])
        if tail_n:
            @pl.when(sid == _NS - 1)
            def _zero_tail():
                pltpu.sync_copy(z_hbm.at[pl.ds(tail_off, tail_n)],
                                aggr_sh.at[pl.ds(tail_off, tail_n)])
        plsc.subcore_barrier()

        for b in range(3):
            si_desc(b, b).start()
        st_desc(0, 0).start()
        si_desc(0, 0).wait()
        for cp in dat_descs(0, 0):
            cp.start()
        si_desc(1, 1).wait()
        for cp in dat_descs(1, 1):
            cp.start()

        @pl.loop(0, chunks, step=R)
        def grp(g0):
            for b in range(R):
                i = g0 + b
                b1 = (b + 1) % R
                b2 = (b + 2) % R
                b3 = (b + 3) % R
                for cp in dat_descs(i, b):
                    cp.wait()
                Hw = rows[b]
                Ev = evs[b]

                @plsc.parallel_loop(0, C, 1, unroll=8)
                def row(r):
                    for j in range(d // 32):
                        sw = pl.ds(j * 16, 16)
                        s1 = pl.ds(j * 16, 16)
                        s2 = pl.ds(d // 2 + j * 16, 16)
                        w = Hw[r, sw]
                        Ev[r, s1] = jnp.maximum(_f32_lo(w) + Ev[r, s1], 0.0)
                        Ev[r, s2] = jnp.maximum(_f32_hi(w) + Ev[r, s2], 0.0)

                st_desc(i, b).wait()
                pltpu.async_copy(Ev, aggr_sh.at[dvs[b]], sss[b], add=True)

                @pl.when(i >= 2)
                def _wait_prev_scatter():
                    sca_desc(b2).wait()

                @pl.when(i + 2 < chunks)
                def _fire_data_ahead():
                    si_desc(i + 2, b2).wait()
                    for cp in dat_descs(i + 2, b2):
                        cp.start()

                @pl.when(i + 3 < chunks)
                def _fire_src_idx():
                    si_desc(i + 3, b3).start()

                @pl.when(i + 1 < chunks)
                def _fire_dst_idx():
                    st_desc(i + 1, b1).start()

        sca_desc((chunks - 2) % R).wait()
        sca_desc((chunks - 1) % R).wait()

        for t in range(tail_e // 16):
            toff = chunks * C + t * 16
            pltpu.sync_copy(src_hbm.at[pl.ds(base + toff, 16)], stt)
            pltpu.sync_copy(dst_hbm.at[pl.ds(base + toff, 16)], dtt)
            pltpu.async_copy(h_hbm.at[stt], rows[0].at[pl.ds(0, 16)],
                             sds[0]).wait()
            pltpu.sync_copy(e_hbm.at[pl.ds(base + toff, 16)],
                            evs[0].at[pl.ds(0, 16)])

            @pl.loop(0, 16)
            def trow(r):
                for j in range(d // 32):
                    sw = pl.ds(j * 16, 16)
                    s1 = pl.ds(j * 16, 16)
                    s2 = pl.ds(d // 2 + j * 16, 16)
                    w = rows[0][r, sw]
                    evs[0][r, s1] = jnp.maximum(_f32_lo(w) + evs[0][r, s1], 0.0)
                    evs[0][r, s2] = jnp.maximum(_f32_hi(w) + evs[0][r, s2], 0.0)

            pltpu.sync_copy(evs[0].at[pl.ds(0, 16)], aggr_sh.at[dtt], add=True)

        plsc.subcore_barrier()
        pltpu.sync_copy(aggr_sh.at[pl.ds(sid * rows_pt, rows_pt)],
                        out_hbm.at[cid, pl.ds(sid * rows_pt, rows_pt)])
        if tail_n:
            @pl.when(sid == _NS - 1)
            def _drain_tail():
                pltpu.sync_copy(aggr_sh.at[pl.ds(tail_off, tail_n)],
                                out_hbm.at[cid, pl.ds(tail_off, tail_n)])

    return sc_aggr


def kernel(node_feats, edge_feats, edge_index, W_proj, b_proj, We1, be1,
           We2, be2, Wc0a, bc0a, Wc0b, bc0b, Wc1a, bc1a, Wc1b, bc1b):
    n, d = node_feats.shape
    e_cnt = edge_feats.shape[0]
    src = edge_index[0]
    dst = edge_index[1]
    zeros_nd = jnp.zeros((n, d), jnp.float32)

    h, hp = _proj(node_feats, W_proj, b_proj.reshape(1, d), blk=2000)
    e = _edge_mlp(edge_feats, We1, be1.reshape(1, d), We2, be2.reshape(1, d),
                  blk=2560)

    sc_aggr = _make_sc_aggr(n, d, e_cnt)
    agg = sc_aggr(hp, e, src, dst, zeros_nd)
    h, hp = _layer_mid(h, agg[0], agg[1], Wc0a, bc0a.reshape(1, d),
                       Wc0b, bc0b.reshape(1, d), blk=2000)
    agg = sc_aggr(hp, e, src, dst, zeros_nd)
    return _layer_out(h, agg[0], agg[1], Wc1a, bc1a.reshape(1, d),
                      Wc1b, bc1b.reshape(1, d), blk=2000)


# speedup vs baseline: 5.4441x; 1.0741x over previous
import functools

import jax
import jax.numpy as jnp
from jax import lax
from jax.experimental import pallas as pl
from jax.experimental.pallas import tpu as pltpu
from jax.experimental.pallas import tpu_sc as plsc

_NC = 2
_NS = 16
_C = 64
_R = 3


def _pack_halves(t):
    d2 = t.shape[1] // 2
    ti = jax.lax.bitcast_convert_type(t, jnp.int32)
    lo = jax.lax.shift_right_logical(ti[:, :d2] + jnp.int32(0x8000), 16)
    hi = (ti[:, d2:] + jnp.int32(0x8000)) & jnp.int32(-65536)
    return lo | hi


def _proj_body(x_ref, w_ref, b_ref, o_ref, op_ref):
    t = jnp.dot(x_ref[...], w_ref[...], preferred_element_type=jnp.float32)
    t = jnp.maximum(t + b_ref[...], 0.0)
    o_ref[...] = t
    op_ref[...] = _pack_halves(t)


def _edge_body(ef_ref, w1_ref, b1_ref, w2_ref, b2_ref, o_ref):
    de = w1_ref.shape[0]
    ef = ef_ref[...]
    halves = []
    for p in range(2):
        t = jnp.dot(ef[:, p * de:(p + 1) * de], w1_ref[...],
                    preferred_element_type=jnp.float32)
        t = jnp.maximum(t + b1_ref[...], 0.0)
        o = jnp.dot(t, w2_ref[...], preferred_element_type=jnp.float32) + b2_ref[...]
        halves.append(_pack_halves(o))
    o_ref[...] = jnp.concatenate(halves, axis=1)


def _layer_mid_body(h_ref, a0_ref, a1_ref, wa_ref, ba_ref, wb_ref, bb_ref,
                    o_ref, op_ref):
    t = h_ref[...] + a0_ref[...] + a1_ref[...]
    u = jnp.dot(t, wa_ref[...], preferred_element_type=jnp.float32)
    u = jnp.maximum(u + ba_ref[...], 0.0)
    v = jnp.dot(u, wb_ref[...], preferred_element_type=jnp.float32)
    v = jnp.maximum(v + bb_ref[...], 0.0)
    o_ref[...] = v
    op_ref[...] = _pack_halves(v)


def _layer_out_body(h_ref, a0_ref, a1_ref, wa_ref, ba_ref, wb_ref, bb_ref, o_ref):
    t = h_ref[...] + a0_ref[...] + a1_ref[...]
    u = jnp.dot(t, wa_ref[...], preferred_element_type=jnp.float32)
    u = jnp.maximum(u + ba_ref[...], 0.0)
    v = jnp.dot(u, wb_ref[...], preferred_element_type=jnp.float32)
    o_ref[...] = jnp.maximum(v + bb_ref[...], 0.0)


def _proj(x, w, b, blk):
    n, d = x.shape
    return pl.pallas_call(
        _proj_body,
        grid=(n // blk,),
        in_specs=[
            pl.BlockSpec((blk, d), lambda i: (i, 0)),
            pl.BlockSpec((d, d), lambda i: (0, 0)),
            pl.BlockSpec((1, d), lambda i: (0, 0)),
        ],
        out_specs=[pl.BlockSpec((blk, d), lambda i: (i, 0)),
                   pl.BlockSpec((blk, d // 2), lambda i: (i, 0))],
        out_shape=[jax.ShapeDtypeStruct((n, d), jnp.float32),
                   jax.ShapeDtypeStruct((n, d // 2), jnp.int32)],
    )(x, w, b)


def _edge_mlp(ef, w1, b1, w2, b2, blk):
    e_cnt, de = ef.shape
    d = w1.shape[1]
    ef2 = ef.reshape(e_cnt // 2, 2 * de)
    return pl.pallas_call(
        _edge_body,
        grid=(e_cnt // blk,),
        in_specs=[
            pl.BlockSpec((blk // 2, 2 * de), lambda i: (i, 0)),
            pl.BlockSpec((de, d), lambda i: (0, 0)),
            pl.BlockSpec((1, d), lambda i: (0, 0)),
            pl.BlockSpec((d, d), lambda i: (0, 0)),
            pl.BlockSpec((1, d), lambda i: (0, 0)),
        ],
        out_specs=pl.BlockSpec((blk // 2, d), lambda i: (i, 0)),
        out_shape=jax.ShapeDtypeStruct((e_cnt // 2, d), jnp.int32),
    )(ef2, w1, b1, w2, b2)


def _layer_specs(blk, d):
    return [
        pl.BlockSpec((blk, d), lambda i: (i, 0)),
        pl.BlockSpec((blk, d), lambda i: (i, 0)),
        pl.BlockSpec((blk, d), lambda i: (i, 0)),
        pl.BlockSpec((d, d), lambda i: (0, 0)),
        pl.BlockSpec((1, d), lambda i: (0, 0)),
        pl.BlockSpec((d, d), lambda i: (0, 0)),
        pl.BlockSpec((1, d), lambda i: (0, 0)),
    ]


def _layer_mid(h, a0, a1, wa, ba, wb, bb, blk):
    n, d = h.shape
    return pl.pallas_call(
        _layer_mid_body,
        grid=(n // blk,),
        in_specs=_layer_specs(blk, d),
        out_specs=[pl.BlockSpec((blk, d), lambda i: (i, 0)),
                   pl.BlockSpec((blk, d // 2), lambda i: (i, 0))],
        out_shape=[jax.ShapeDtypeStruct((n, d), jnp.float32),
                   jax.ShapeDtypeStruct((n, d // 2), jnp.int32)],
    )(h, a0, a1, wa, ba, wb, bb)


def _layer_out(h, a0, a1, wa, ba, wb, bb, blk):
    n, d = h.shape
    return pl.pallas_call(
        _layer_out_body,
        grid=(n // blk,),
        in_specs=_layer_specs(blk, d),
        out_specs=pl.BlockSpec((blk, d), lambda i: (i, 0)),
        out_shape=jax.ShapeDtypeStruct((n, d), jnp.float32),
    )(h, a0, a1, wa, ba, wb, bb)


def _f32_lo(w):
    return jax.lax.bitcast_convert_type(w << 16, jnp.float32)


def _f32_hi(w):
    return jax.lax.bitcast_convert_type(w & jnp.int32(-65536), jnp.float32)


@functools.cache
def _make_sc_aggr(n, d, e_cnt):
    nw = _NC * _NS
    epw = e_cnt // nw
    C = _C
    R = _R
    chunks = (epw // C) // R * R
    tail_e = epw - chunks * C
    rows_pt = (n // _NS) // 8 * 8
    tail_off = rows_pt * _NS
    tail_n = n - tail_off
    assert epw * nw == e_cnt and tail_e % 16 == 0
    assert tail_n % 8 == 0 and tail_off % 8 == 0
    mesh = plsc.VectorSubcoreMesh(core_axis_name="c", subcore_axis_name="s")
    vm = pltpu.VMEM
    DMA = pltpu.SemaphoreType.DMA

    @functools.partial(
        pl.kernel,
        out_type=jax.ShapeDtypeStruct((_NC, n, d), jnp.float32),
        mesh=mesh,
        compiler_params=pltpu.CompilerParams(use_tc_tiling_on_sc=False),
        scratch_types=[
            [vm((C,), jnp.int32)] * R,
            [vm((C,), jnp.int32)] * R,
            [vm((C, d // 2), jnp.int32)] * R,
            [vm((C // 2, d), jnp.int32)] * R,
            [vm((C, d), jnp.float32)] * R,
            vm((16,), jnp.int32),
            vm((16,), jnp.int32),
            pltpu.VMEM_SHARED((n, d), jnp.float32),
            [DMA] * R,
            [DMA] * R,
            [DMA] * R,
            [DMA] * R,
        ],
    )
    def sc_aggr(h_hbm, e_hbm, src_hbm, dst_hbm, z_hbm, out_hbm,
                svs, dvs, rows, evs, ms, stt, dtt, aggr_sh, sis, sts, sds, sss):
        cid = lax.axis_index("c")
        sid = lax.axis_index("s")
        wid = cid * _NS + sid
        base = wid * epw

        def si_desc(i, b):
            return pltpu.make_async_copy(src_hbm.at[pl.ds(base + i * C, C)],
                                         svs[b], sis[b])

        def st_desc(i, b):
            return pltpu.make_async_copy(dst_hbm.at[pl.ds(base + i * C, C)],
                                         dvs[b], sts[b])

        def dat_descs(i, b):
            g = pltpu.make_async_copy(h_hbm.at[svs[b]], rows[b], sds[b])
            s = pltpu.make_async_copy(
                e_hbm.at[pl.ds((base + i * C) // 2, C // 2)], evs[b], sds[b])
            return g, s

        def sca_desc(b):
            return pltpu.make_async_copy(ms[b], aggr_sh.at[dvs[b]], sss[b])

        pltpu.sync_copy(z_hbm.at[pl.ds(sid * rows_pt, rows_pt)],
                        aggr_sh.at[pl.ds(sid * rows_pt, rows_pt)])
        if tail_n:
            @pl.when(sid == _NS - 1)
            def _zero_tail():
                pltpu.sync_copy(z_hbm.at[pl.ds(tail_off, tail_n)],
                                aggr_sh.at[pl.ds(tail_off, tail_n)])
        plsc.subcore_barrier()

        for b in range(3):
            si_desc(b, b).start()
        st_desc(0, 0).start()
        si_desc(0, 0).wait()
        for cp in dat_descs(0, 0):
            cp.start()
        si_desc(1, 1).wait()
        for cp in dat_descs(1, 1):
            cp.start()

        @pl.loop(0, chunks, step=R)
        def grp(g0):
            for b in range(R):
                i = g0 + b
                b1 = (b + 1) % R
                b2 = (b + 2) % R
                b3 = (b + 3) % R
                for cp in dat_descs(i, b):
                    cp.wait()
                Hw = rows[b]
                Ev = evs[b]
                M = ms[b]

                @plsc.parallel_loop(0, C // 2, 1, unroll=4)
                def pair(q):
                    for p in range(2):
                        r = 2 * q + p
                        for j in range(d // 32):
                            wh = Hw[r, pl.ds(j * 16, 16)]
                            we = Ev[q, pl.ds(p * (d // 2) + j * 16, 16)]
                            M[r, pl.ds(j * 16, 16)] = jnp.maximum(
                                _f32_lo(wh) + _f32_lo(we), 0.0)
                            M[r, pl.ds(d // 2 + j * 16, 16)] = jnp.maximum(
                                _f32_hi(wh) + _f32_hi(we), 0.0)

                st_desc(i, b).wait()
                pltpu.async_copy(M, aggr_sh.at[dvs[b]], sss[b], add=True)

                @pl.when(i >= R - 2)
                def _wait_prev_scatter():
                    sca_desc(b2).wait()

                @pl.when(i + 2 < chunks)
                def _fire_data_ahead():
                    si_desc(i + 2, b2).wait()
                    for cp in dat_descs(i + 2, b2):
                        cp.start()

                @pl.when(i + 3 < chunks)
                def _fire_src_idx():
                    si_desc(i + 3, b3).start()

                @pl.when(i + 1 < chunks)
                def _fire_dst_idx():
                    st_desc(i + 1, b1).start()

        for k in range(R - 2, 0, -1):
            sca_desc((chunks - k) % R).wait()

        for t in range(tail_e // 16):
            toff = chunks * C + t * 16
            pltpu.sync_copy(src_hbm.at[pl.ds(base + toff, 16)], stt)
            pltpu.sync_copy(dst_hbm.at[pl.ds(base + toff, 16)], dtt)
            pltpu.async_copy(h_hbm.at[stt], rows[0].at[pl.ds(0, 16)],
                             sds[0]).wait()
            pltpu.sync_copy(e_hbm.at[pl.ds((base + toff) // 2, 8)],
                            evs[0].at[pl.ds(0, 8)])

            @pl.loop(0, 8)
            def tpair(q):
                for p in range(2):
                    r = 2 * q + p
                    for j in range(d // 32):
                        wh = rows[0][r, pl.ds(j * 16, 16)]
                        we = evs[0][q, pl.ds(p * (d // 2) + j * 16, 16)]
                        ms[0][r, pl.ds(j * 16, 16)] = jnp.maximum(
                            _f32_lo(wh) + _f32_lo(we), 0.0)
                        ms[0][r, pl.ds(d // 2 + j * 16, 16)] = jnp.maximum(
                            _f32_hi(wh) + _f32_hi(we), 0.0)

            pltpu.sync_copy(ms[0].at[pl.ds(0, 16)], aggr_sh.at[dtt], add=True)

        plsc.subcore_barrier()
        pltpu.sync_copy(aggr_sh.at[pl.ds(sid * rows_pt, rows_pt)],
                        out_hbm.at[cid, pl.ds(sid * rows_pt, rows_pt)])
        if tail_n:
            @pl.when(sid == _NS - 1)
            def _drain_tail():
                pltpu.sync_copy(aggr_sh.at[pl.ds(tail_off, tail_n)],
                                out_hbm.at[cid, pl.ds(tail_off, tail_n)])

    return sc_aggr


def kernel(node_feats, edge_feats, edge_index, W_proj, b_proj, We1, be1,
           We2, be2, Wc0a, bc0a, Wc0b, bc0b, Wc1a, bc1a, Wc1b, bc1b):
    n, d = node_feats.shape
    e_cnt = edge_feats.shape[0]
    src = edge_index[0]
    dst = edge_index[1]
    zeros_nd = jnp.zeros((n, d), jnp.float32)

    h, hp = _proj(node_feats, W_proj, b_proj.reshape(1, d), blk=2000)
    e = _edge_mlp(edge_feats, We1, be1.reshape(1, d), We2, be2.reshape(1, d),
                  blk=2560)

    sc_aggr = _make_sc_aggr(n, d, e_cnt)
    agg = sc_aggr(hp, e, src, dst, zeros_nd)
    h, hp = _layer_mid(h, agg[0], agg[1], Wc0a, bc0a.reshape(1, d),
                       Wc0b, bc0b.reshape(1, d), blk=2000)
    agg = sc_aggr(hp, e, src, dst, zeros_nd)
    return _layer_out(h, agg[0], agg[1], Wc1a, bc1a.reshape(1, d),
                      Wc1b, bc1b.reshape(1, d), blk=2000)
